# Initial kernel scaffold; baseline (speedup 1.0000x reference)
import jax
import jax.numpy as jnp
from jax.experimental import pallas as pl


def kernel(query, reference_points, input_flatten, input_spatial_shapes, W_off, b_off, W_attn, b_attn, W_val, b_val):
    raise NotImplementedError("write your pallas kernel here")


# speedup vs baseline: 229.1260x; 229.1260x over previous
import functools

import jax
import jax.numpy as jnp
from jax import lax
from jax.experimental import pallas as pl
from jax.experimental.pallas import tpu as pltpu
from jax.experimental.pallas import tpu_sc as plsc

_D = 256
_NH = 8
_NL = 4
_NP = 8
_DH = 32
_SHAPES = ((116, 200), (58, 100), (29, 50), (15, 25))
_S = sum(h * w for h, w in _SHAPES)
_LQ = 10000
_LQP = 10240
_SP = 30848
_NW = 32
_QPW = _LQP // _NW


def _vproj_body(x_ref, wt_ref, b_ref, o_ref):
    o_ref[...] = jnp.dot(x_ref[...], wt_ref[...],
                         preferred_element_type=jnp.float32) + b_ref[...]


def _qproj_body(q_ref, wofft_ref, boff_ref, wattnt_ref, battn_ref,
                off_ref, attn_ref):
    q = q_ref[...]
    off_ref[...] = jnp.dot(q, wofft_ref[...],
                           preferred_element_type=jnp.float32) + boff_ref[...]
    lg = jnp.dot(q, wattnt_ref[...],
                 preferred_element_type=jnp.float32) + battn_ref[...]
    l3 = lg.reshape(q.shape[0], _NH, _NL * _NP)
    m = jnp.max(l3, axis=-1, keepdims=True)
    e = jnp.exp(l3 - m)
    s = jnp.sum(e, axis=-1, keepdims=True)
    attn_ref[...] = (e / s).reshape(q.shape[0], _NH * _NL * _NP)


def _sc_gather_reduce(table, idxs, wtss):
    info = plsc.get_sparse_core_info()
    nc = info.num_cores

    @functools.partial(
        pl.kernel,
        mesh=plsc.VectorSubcoreMesh(core_axis_name="c", subcore_axis_name="s"),
        out_type=jax.ShapeDtypeStruct((_LQP, _D), jnp.float32),
        scratch_types=[
            pltpu.VMEM((128,), jnp.int32),
            pltpu.VMEM((128, 128), jnp.float32),
            pltpu.VMEM((128, 16), jnp.float32),
            pltpu.VMEM((_D,), jnp.float32),
            pltpu.SemaphoreType.DMA,
        ],
    )
    def k(table_hbm, idx_hbm, wts_hbm, out_hbm, idx_v, rows_v, wts_v, out_v, sem):
        wid = lax.axis_index("s") * nc + lax.axis_index("c")

        def per_q(t, carry):
            q = wid * _QPW + t
            for h in range(_NH):
                co = (h % 4) * _DH
                pltpu.sync_copy(idx_hbm.at[q, h], idx_v)
                pltpu.sync_copy(wts_hbm.at[q, h], wts_v)
                pltpu.async_copy(table_hbm.at[idx_v], rows_v, sem).wait()

                def red(j, acc):
                    a0, a1 = acc
                    wv = wts_v[j, pl.ds(0, 16)]
                    r0 = rows_v[j, pl.ds(co, 16)]
                    r1 = rows_v[j, pl.ds(co + 16, 16)]
                    return (a0 + wv * r0, a1 + wv * r1)

                z = jnp.zeros((16,), jnp.float32)
                a0, a1 = lax.fori_loop(0, 128, red, (z, z))
                out_v[pl.ds(h * _DH, 16)] = a0
                out_v[pl.ds(h * _DH + 16, 16)] = a1
            pltpu.sync_copy(out_v, out_hbm.at[q])
            return carry

        lax.fori_loop(0, _QPW, per_q, 0)

    return k(table, idxs, wtss)


def kernel(query, reference_points, input_flatten, input_spatial_shapes,
           W_off, b_off, W_attn, b_attn, W_val, b_val):
    f32 = jnp.float32

    x = jnp.zeros((_SP, _D), f32).at[:_S].set(input_flatten[0])
    value = pl.pallas_call(
        _vproj_body,
        grid=(_SP // 128,),
        in_specs=[
            pl.BlockSpec((128, _D), lambda i: (i, 0)),
            pl.BlockSpec((_D, _D), lambda i: (0, 0)),
            pl.BlockSpec((1, _D), lambda i: (0, 0)),
        ],
        out_specs=pl.BlockSpec((128, _D), lambda i: (i, 0)),
        out_shape=jax.ShapeDtypeStruct((_SP, _D), f32),
    )(x, W_val.T, b_val.reshape(1, _D))
    table = value[:_S].reshape(_S * 2, 128)

    qx = jnp.zeros((_LQP, _D), f32).at[:_LQ].set(query[0])
    noff = _NH * _NL * _NP * 2
    nattn = _NH * _NL * _NP
    off_flat, attn_flat = pl.pallas_call(
        _qproj_body,
        grid=(_LQP // 256,),
        in_specs=[
            pl.BlockSpec((256, _D), lambda i: (i, 0)),
            pl.BlockSpec((_D, noff), lambda i: (0, 0)),
            pl.BlockSpec((1, noff), lambda i: (0, 0)),
            pl.BlockSpec((_D, nattn), lambda i: (0, 0)),
            pl.BlockSpec((1, nattn), lambda i: (0, 0)),
        ],
        out_specs=[
            pl.BlockSpec((256, noff), lambda i: (i, 0)),
            pl.BlockSpec((256, nattn), lambda i: (i, 0)),
        ],
        out_shape=[
            jax.ShapeDtypeStruct((_LQP, noff), f32),
            jax.ShapeDtypeStruct((_LQP, nattn), f32),
        ],
    )(qx, W_off.T, b_off.reshape(1, noff), W_attn.T, b_attn.reshape(1, nattn))

    nz = reference_points.shape[2]
    norm = input_spatial_shapes[:, ::-1].astype(f32)
    off = off_flat[:_LQ].reshape(_LQ, _NH, _NL, _NP // nz, nz, 2)
    off = off / norm[None, None, :, None, None, :]
    loc = reference_points[0][:, None, None, None, :, :] + off
    loc = loc.reshape(_LQ, _NH, _NL, _NP, 2)
    attn = attn_flat[:_LQ].reshape(_LQ, _NH, _NL, _NP)

    Wv = jnp.asarray([w for _, w in _SHAPES], f32)[None, None, :, None]
    Hv = jnp.asarray([h for h, _ in _SHAPES], f32)[None, None, :, None]
    bases = []
    acc = 0
    for h_, w_ in _SHAPES:
        bases.append(acc)
        acc += h_ * w_
    base = jnp.asarray(bases, jnp.int32)[None, None, :, None]
    Wi = Wv.astype(jnp.int32)
    gx = loc[..., 0] * Wv - 0.5
    gy = loc[..., 1] * Hv - 0.5
    x0 = jnp.floor(gx)
    y0 = jnp.floor(gy)
    fx = gx - x0
    fy = gy - y0
    hidx = jnp.arange(_NH, dtype=jnp.int32)[None, :, None, None]

    idx_c, wts_c = [], []
    for dx, dy in ((0, 0), (1, 0), (0, 1), (1, 1)):
        xc = x0 + dx
        yc = y0 + dy
        valid = ((xc >= 0) & (xc <= Wv - 1) & (yc >= 0) & (yc <= Hv - 1))
        xi = jnp.clip(xc, 0, Wv - 1).astype(jnp.int32)
        yi = jnp.clip(yc, 0, Hv - 1).astype(jnp.int32)
        row = (base + yi * Wi + xi) * 2 + hidx // 4
        wgt = ((fx if dx else 1.0 - fx) * (fy if dy else 1.0 - fy)
               * valid.astype(f32) * attn)
        idx_c.append(row)
        wts_c.append(wgt)
    idx = jnp.stack(idx_c, axis=-1).reshape(_LQ, _NH, _NL * _NP * 4)
    wts = jnp.stack(wts_c, axis=-1).reshape(_LQ, _NH, _NL * _NP * 4)
    idx = jnp.zeros((_LQP, _NH, 128), jnp.int32).at[:_LQ].set(idx)
    wts = jnp.zeros((_LQP, _NH, 128), f32).at[:_LQ].set(wts)
    wts = jnp.broadcast_to(wts[..., None], (_LQP, _NH, 128, 16))

    out = _sc_gather_reduce(table, idx, wts)
    return out[:_LQ][None]


# speedup vs baseline: 269.2982x; 1.1753x over previous
import functools

import jax
import jax.numpy as jnp
from jax import lax
from jax.experimental import pallas as pl
from jax.experimental.pallas import tpu as pltpu
from jax.experimental.pallas import tpu_sc as plsc

_D = 256
_NH = 8
_NL = 4
_NP = 8
_DH = 32
_SHAPES = ((116, 200), (58, 100), (29, 50), (15, 25))
_S = sum(h * w for h, w in _SHAPES)
_LQ = 10000
_LQP = 10240
_SP = 30848
_NW = 32
_QPW = _LQP // _NW


def _vproj_body(x_ref, wt_ref, b_ref, o_ref):
    o_ref[...] = jnp.dot(x_ref[...], wt_ref[...],
                         preferred_element_type=jnp.float32) + b_ref[...]


def _qproj_body(q_ref, wofft_ref, boff_ref, wattnt_ref, battn_ref,
                off_ref, attn_ref):
    q = q_ref[...]
    off_ref[...] = jnp.dot(q, wofft_ref[...],
                           preferred_element_type=jnp.float32) + boff_ref[...]
    lg = jnp.dot(q, wattnt_ref[...],
                 preferred_element_type=jnp.float32) + battn_ref[...]
    l3 = lg.reshape(q.shape[0], _NH, _NL * _NP)
    m = jnp.max(l3, axis=-1, keepdims=True)
    e = jnp.exp(l3 - m)
    s = jnp.sum(e, axis=-1, keepdims=True)
    attn_ref[...] = (e / s).reshape(q.shape[0], _NH * _NL * _NP)


def _sc_gather_reduce(table, idxs, wtss):
    info = plsc.get_sparse_core_info()
    nc = info.num_cores

    @functools.partial(
        pl.kernel,
        mesh=plsc.VectorSubcoreMesh(core_axis_name="c", subcore_axis_name="s"),
        out_type=jax.ShapeDtypeStruct((_LQP, _D), jnp.float32),
        scratch_types=[
            pltpu.VMEM((128,), jnp.int32),
            pltpu.VMEM((128,), jnp.int32),
            pltpu.VMEM((128, 128), jnp.float32),
            pltpu.VMEM((128, 128), jnp.float32),
            pltpu.VMEM((128, 16), jnp.float32),
            pltpu.VMEM((128, 16), jnp.float32),
            pltpu.VMEM((_D,), jnp.float32),
            pltpu.SemaphoreType.DMA,
            pltpu.SemaphoreType.DMA,
        ],
    )
    def k(table_hbm, idx_hbm, wts_hbm, out_hbm,
          idx_a, idx_b, rows_a, rows_b, wts_a, wts_b, out_v, sem_a, sem_b):
        wid = lax.axis_index("s") * nc + lax.axis_index("c")
        idx_bufs = (idx_a, idx_b)
        row_bufs = (rows_a, rows_b)
        wts_bufs = (wts_a, wts_b)
        sems = (sem_a, sem_b)

        def reduce_head(h, rows_v, wts_v):
            co = (h % 4) * _DH

            def red(j, acc):
                a0, a1 = acc
                wv = wts_v[j, pl.ds(0, 16)]
                r0 = rows_v[j, pl.ds(co, 16)]
                r1 = rows_v[j, pl.ds(co + 16, 16)]
                return (a0 + wv * r0, a1 + wv * r1)

            z = jnp.zeros((16,), jnp.float32)
            a0, a1 = lax.fori_loop(0, 128, red, (z, z))
            out_v[pl.ds(h * _DH, 16)] = a0
            out_v[pl.ds(h * _DH + 16, 16)] = a1

        def per_q(t, carry):
            q = wid * _QPW + t
            pltpu.sync_copy(idx_hbm.at[q, 0], idx_bufs[0])
            pltpu.sync_copy(wts_hbm.at[q, 0], wts_bufs[0])
            cp = pltpu.async_copy(table_hbm.at[idx_bufs[0]], row_bufs[0],
                                  sems[0])
            for h in range(1, _NH):
                b, pb = h % 2, (h - 1) % 2
                pltpu.sync_copy(idx_hbm.at[q, h], idx_bufs[b])
                pltpu.sync_copy(wts_hbm.at[q, h], wts_bufs[b])
                nxt = pltpu.async_copy(table_hbm.at[idx_bufs[b]], row_bufs[b],
                                       sems[b])
                cp.wait()
                reduce_head(h - 1, row_bufs[pb], wts_bufs[pb])
                cp = nxt
            cp.wait()
            reduce_head(_NH - 1, row_bufs[(_NH - 1) % 2], wts_bufs[(_NH - 1) % 2])
            pltpu.sync_copy(out_v, out_hbm.at[q])
            return carry

        lax.fori_loop(0, _QPW, per_q, 0)

    return k(table, idxs, wtss)


def kernel(query, reference_points, input_flatten, input_spatial_shapes,
           W_off, b_off, W_attn, b_attn, W_val, b_val):
    f32 = jnp.float32

    x = jnp.zeros((_SP, _D), f32).at[:_S].set(input_flatten[0])
    value = pl.pallas_call(
        _vproj_body,
        grid=(_SP // 128,),
        in_specs=[
            pl.BlockSpec((128, _D), lambda i: (i, 0)),
            pl.BlockSpec((_D, _D), lambda i: (0, 0)),
            pl.BlockSpec((1, _D), lambda i: (0, 0)),
        ],
        out_specs=pl.BlockSpec((128, _D), lambda i: (i, 0)),
        out_shape=jax.ShapeDtypeStruct((_SP, _D), f32),
    )(x, W_val.T, b_val.reshape(1, _D))
    table = value[:_S].reshape(_S * 2, 128)

    qx = jnp.zeros((_LQP, _D), f32).at[:_LQ].set(query[0])
    noff = _NH * _NL * _NP * 2
    nattn = _NH * _NL * _NP
    off_flat, attn_flat = pl.pallas_call(
        _qproj_body,
        grid=(_LQP // 256,),
        in_specs=[
            pl.BlockSpec((256, _D), lambda i: (i, 0)),
            pl.BlockSpec((_D, noff), lambda i: (0, 0)),
            pl.BlockSpec((1, noff), lambda i: (0, 0)),
            pl.BlockSpec((_D, nattn), lambda i: (0, 0)),
            pl.BlockSpec((1, nattn), lambda i: (0, 0)),
        ],
        out_specs=[
            pl.BlockSpec((256, noff), lambda i: (i, 0)),
            pl.BlockSpec((256, nattn), lambda i: (i, 0)),
        ],
        out_shape=[
            jax.ShapeDtypeStruct((_LQP, noff), f32),
            jax.ShapeDtypeStruct((_LQP, nattn), f32),
        ],
    )(qx, W_off.T, b_off.reshape(1, noff), W_attn.T, b_attn.reshape(1, nattn))

    nz = reference_points.shape[2]
    norm = input_spatial_shapes[:, ::-1].astype(f32)
    off = off_flat[:_LQ].reshape(_LQ, _NH, _NL, _NP // nz, nz, 2)
    off = off / norm[None, None, :, None, None, :]
    loc = reference_points[0][:, None, None, None, :, :] + off
    loc = loc.reshape(_LQ, _NH, _NL, _NP, 2)
    attn = attn_flat[:_LQ].reshape(_LQ, _NH, _NL, _NP)

    Wv = jnp.asarray([w for _, w in _SHAPES], f32)[None, None, :, None]
    Hv = jnp.asarray([h for h, _ in _SHAPES], f32)[None, None, :, None]
    bases = []
    acc = 0
    for h_, w_ in _SHAPES:
        bases.append(acc)
        acc += h_ * w_
    base = jnp.asarray(bases, jnp.int32)[None, None, :, None]
    Wi = Wv.astype(jnp.int32)
    gx = loc[..., 0] * Wv - 0.5
    gy = loc[..., 1] * Hv - 0.5
    x0 = jnp.floor(gx)
    y0 = jnp.floor(gy)
    fx = gx - x0
    fy = gy - y0
    hidx = jnp.arange(_NH, dtype=jnp.int32)[None, :, None, None]

    idx_c, wts_c = [], []
    for dx, dy in ((0, 0), (1, 0), (0, 1), (1, 1)):
        xc = x0 + dx
        yc = y0 + dy
        valid = ((xc >= 0) & (xc <= Wv - 1) & (yc >= 0) & (yc <= Hv - 1))
        xi = jnp.clip(xc, 0, Wv - 1).astype(jnp.int32)
        yi = jnp.clip(yc, 0, Hv - 1).astype(jnp.int32)
        row = (base + yi * Wi + xi) * 2 + hidx // 4
        wgt = ((fx if dx else 1.0 - fx) * (fy if dy else 1.0 - fy)
               * valid.astype(f32) * attn)
        idx_c.append(row)
        wts_c.append(wgt)
    idx = jnp.stack(idx_c, axis=-1).reshape(_LQ, _NH, _NL * _NP * 4)
    wts = jnp.stack(wts_c, axis=-1).reshape(_LQ, _NH, _NL * _NP * 4)
    idx = jnp.zeros((_LQP, _NH, 128), jnp.int32).at[:_LQ].set(idx)
    wts = jnp.zeros((_LQP, _NH, 128), f32).at[:_LQ].set(wts)
    wts = jnp.broadcast_to(wts[..., None], (_LQP, _NH, 128, 16))

    out = _sc_gather_reduce(table, idx, wts)
    return out[:_LQ][None]
